# baseline (speedup 1.0000x reference)
import functools

import jax
import jax.numpy as jnp
from jax import lax
from jax.experimental import pallas as pl
from jax.experimental.pallas import tpu as pltpu
from jax.experimental.pallas import tpu_sc as plsc

_GRID = (128, 128, 128)
_N = 50000
_NW = 32
_RPW = 1600
_NP = _NW * _RPW
_C = 128
_K = 9
_OC = 320
_NOC = _RPW // _OC
_NB = _NW * _NOC
_BKT = _OC * _K
_PCH = 128
_CAPW = 3072
_W2 = 5888
_BM = 512
_NPB = _NP // _BM

_OFF_A = tuple((0, dy, dx) for dy in (-1, 0, 1) for dx in (-1, 0, 1))
_OFF_B = tuple((dz, 0, dx) for dz in (-1, 0, 1) for dx in (-1, 0, 1))


def _nbr(grid, coords, offsets):
    D, H, Wd = _GRID
    z, y, x = coords[:, 0], coords[:, 1], coords[:, 2]
    cols = []
    for oz, oy, ox in offsets:
        nz, ny, nx = z + oz, y + oy, x + ox
        valid = (nz >= 0) & (nz < D) & (ny >= 0) & (ny < H) & (nx >= 0) & (nx < Wd)
        nb = grid[jnp.clip(nz, 0, D - 1), jnp.clip(ny, 0, H - 1), jnp.clip(nx, 0, Wd - 1)]
        cols.append(jnp.where(valid & (nb >= 0), nb, _N))
    nb = jnp.stack(cols)
    pad = jnp.full((_K, _NP - _N), _N, jnp.int32)
    return jnp.concatenate([nb, pad], axis=1)


def _pairs(nb):
    nbt = nb.T
    ii = jnp.arange(_NP, dtype=jnp.int32)[:, None]
    kk = jnp.arange(_K, dtype=jnp.int32)[None, :]
    need = (nbt != _N) & ((kk != 4) | (nbt != ii))
    src = nbt * _K + kk
    dst = jnp.broadcast_to(ii % _OC, (_NP, _K))
    needb = need.reshape(_NB, _BKT)
    e = jnp.cumsum(needb.astype(jnp.int32), axis=1) - needb
    posb = jnp.broadcast_to(jnp.arange(_BKT, dtype=jnp.int32)[None, :],
                            (_NB, _BKT))
    col = jnp.where(needb, e, 2944 + posb - e)
    flatpos = (jnp.arange(_NB, dtype=jnp.int32)[:, None] * _W2 + col).ravel()
    srcs = jnp.full((_NB * _W2,), _N * _K, jnp.int32).at[flatpos].set(
        src.reshape(-1), unique_indices=True, mode="promise_in_bounds")
    dsts = jnp.zeros((_NB * _W2,), jnp.int32).at[flatpos].set(
        dst.reshape(-1), unique_indices=True, mode="promise_in_bounds")
    srcs = srcs.reshape(_NB, _W2)[:, :_CAPW]
    dsts = dsts.reshape(_NB, _W2)[:, :_CAPW]
    counts = jnp.sum(needb, axis=1, dtype=jnp.int32).reshape(_NW, _NOC)
    counts = jnp.pad(counts, ((0, 0), (0, 16 - _NOC)))
    return (srcs.reshape(_NB, 1, _CAPW), dsts.reshape(_NB, 1, _CAPW),
            counts.reshape(_NW, 1, 16))


def _cat9(W):
    return W.reshape(_K, _C, _C).transpose(1, 0, 2).reshape(_C, _K * _C)


def _mm_plain_body(x_ref, w_ref, m_ref, y_ref, b0_ref):
    y = jnp.dot(x_ref[...], w_ref[...], preferred_element_type=jnp.float32)
    y_ref[...] = y
    b0_ref[...] = y[:, 4 * _C:5 * _C] * m_ref[...]


def _mm_plain(x, wcat, mult):
    return pl.pallas_call(
        _mm_plain_body,
        grid=(_NPB,),
        in_specs=[pl.BlockSpec((_BM, _C), lambda i: (i, 0)),
                  pl.BlockSpec((_C, _K * _C), lambda i: (0, 0)),
                  pl.BlockSpec((_BM, 1), lambda i: (i, 0))],
        out_specs=[pl.BlockSpec((_BM, _K * _C), lambda i: (i, 0)),
                   pl.BlockSpec((_BM, _C), lambda i: (i, 0))],
        out_shape=[jax.ShapeDtypeStruct((_NP, _K * _C), jnp.float32),
                   jax.ShapeDtypeStruct((_NP, _C), jnp.float32)],
    )(x, wcat, mult)


def _bn_affine(p, bnw, bnb):
    psum = jnp.sum(p, axis=0)
    mu = psum[0:1] / float(_N)
    msq = psum[1:2] / float(_N)
    var = msq - mu * mu
    s = bnw * lax.rsqrt(var + 1e-5)
    t = bnb - mu * s
    return s, t


def _mm_fused_body(z_ref, w_ref, bnw_ref, bnb_ref, p_ref, m_ref, y_ref, b0_ref):
    s, t = _bn_affine(p_ref[...], bnw_ref[...], bnb_ref[...])
    z = z_ref[...]
    l = jnp.where(z >= 0.0, z, z * 0.01)
    rows = pl.program_id(0) * _BM + lax.broadcasted_iota(jnp.int32, (_BM, 1), 0)
    a = jnp.where(rows < _N, l * s + t, 0.0)
    y = jnp.dot(a, w_ref[...], preferred_element_type=jnp.float32)
    y_ref[...] = y
    b0_ref[...] = y[:, 4 * _C:5 * _C] * m_ref[...]


def _mm_fused(z, wcat, bnw, bnb, p, mult):
    return pl.pallas_call(
        _mm_fused_body,
        grid=(_NPB,),
        in_specs=[pl.BlockSpec((_BM, _C), lambda i: (i, 0)),
                  pl.BlockSpec((_C, _K * _C), lambda i: (0, 0)),
                  pl.BlockSpec((1, _C), lambda i: (0, 0)),
                  pl.BlockSpec((1, _C), lambda i: (0, 0)),
                  pl.BlockSpec((_NPB, 2, _C), lambda i: (0, 0, 0)),
                  pl.BlockSpec((_BM, 1), lambda i: (i, 0))],
        out_specs=[pl.BlockSpec((_BM, _K * _C), lambda i: (i, 0)),
                   pl.BlockSpec((_BM, _C), lambda i: (i, 0))],
        out_shape=[jax.ShapeDtypeStruct((_NP, _K * _C), jnp.float32),
                   jax.ShapeDtypeStruct((_NP, _C), jnp.float32)],
    )(z, wcat, bnw, bnb, p, mult)


def _stats_body(z_ref, p_ref):
    z = z_ref[...]
    l = jnp.where(z >= 0.0, z, z * 0.01)
    p_ref[...] = jnp.stack([jnp.sum(l, axis=0), jnp.sum(l * l, axis=0)])[None]


def _stats(z):
    return pl.pallas_call(
        _stats_body,
        grid=(_NPB,),
        in_specs=[pl.BlockSpec((_BM, _C), lambda i: (i, 0))],
        out_specs=pl.BlockSpec((1, 2, _C), lambda i: (i, 0, 0)),
        out_shape=jax.ShapeDtypeStruct((_NPB, 2, _C), jnp.float32),
    )(z)


def _final_body(z2_ref, z4_ref, p2_ref, p4_ref, w2_ref, b2_ref, w4_ref, b4_ref,
                o_ref):
    s2, t2 = _bn_affine(p2_ref[...], w2_ref[...], b2_ref[...])
    s4, t4 = _bn_affine(p4_ref[...], w4_ref[...], b4_ref[...])
    z2 = z2_ref[...]
    z4 = z4_ref[...]
    l2 = jnp.where(z2 >= 0.0, z2, z2 * 0.01)
    l4 = jnp.where(z4 >= 0.0, z4, z4 * 0.01)
    o_ref[...] = (l2 * s2 + t2) + (l4 * s4 + t4)


def _final(z2, p2, w2, b2, z4, p4, w4, b4):
    return pl.pallas_call(
        _final_body,
        grid=(_NPB,),
        in_specs=[pl.BlockSpec((_BM, _C), lambda i: (i, 0)),
                  pl.BlockSpec((_BM, _C), lambda i: (i, 0)),
                  pl.BlockSpec((_NPB, 2, _C), lambda i: (0, 0, 0)),
                  pl.BlockSpec((_NPB, 2, _C), lambda i: (0, 0, 0)),
                  pl.BlockSpec((1, _C), lambda i: (0, 0)),
                  pl.BlockSpec((1, _C), lambda i: (0, 0)),
                  pl.BlockSpec((1, _C), lambda i: (0, 0)),
                  pl.BlockSpec((1, _C), lambda i: (0, 0))],
        out_specs=pl.BlockSpec((_BM, _C), lambda i: (i, 0)),
        out_shape=jax.ShapeDtypeStruct((_NP, _C), jnp.float32),
    )(z2, z4, p2, p4, w2, b2, w4, b4)


def _gather_body(tab_hbm, b0_hbm, srcs_hbm, dsts_hbm, cnts_hbm, z_hbm,
                 srcs_v, rows_v, out_v, sh_cnt, sh_dst, cnt_sm, dst_sm,
                 semg, sw0, sw1):
    sid = lax.axis_index("s")
    wid = sid * 2 + lax.axis_index("c")
    pltpu.sync_copy(cnts_hbm.at[wid], sh_cnt.at[sid])
    pltpu.sync_copy(sh_cnt.at[sid], cnt_sm)
    sw = (sw0, sw1)
    for oc in range(_NOC):
        par = oc % 2
        obase = wid * _RPW + oc * _OC
        if oc >= 2:
            pltpu.make_async_copy(out_v.at[par], z_hbm.at[pl.ds(0, _OC)],
                                  sw[par]).wait()
        pltpu.sync_copy(b0_hbm.at[pl.ds(obase, _OC)], out_v.at[par])
        n = cnt_sm[0, oc]
        nch = lax.div(n + _PCH - 1, _PCH)

        def chunk(cc, carry, oc=oc, par=par):
            pltpu.sync_copy(srcs_hbm.at[wid * _NOC + oc, 0, pl.ds(cc * _PCH, _PCH)], srcs_v)
            pltpu.sync_copy(dsts_hbm.at[wid * _NOC + oc, 0, pl.ds(cc * _PCH, _PCH)],
                            sh_dst.at[sid])
            pltpu.sync_copy(sh_dst.at[sid], dst_sm)
            pltpu.async_copy(tab_hbm.at[srcs_v], rows_v, semg).wait()

            def pr(j, c2):
                r = dst_sm[j]
                for c in range(8):
                    sl = pl.ds(c * 16, 16)
                    plsc.addupdate(out_v.at[par, r, sl], rows_v[j, sl])
                return c2

            lax.fori_loop(0, _PCH, pr, 0)
            return carry

        lax.fori_loop(0, nch, chunk, 0)
        pltpu.async_copy(out_v.at[par], z_hbm.at[pl.ds(obase, _OC)], sw[par])
    pltpu.make_async_copy(out_v.at[1], z_hbm.at[pl.ds(0, _OC)], sw1).wait()
    pltpu.make_async_copy(out_v.at[0], z_hbm.at[pl.ds(0, _OC)], sw0).wait()


def _gather_sc(tab, b0, srcs, dsts, cnts):
    mesh = plsc.VectorSubcoreMesh(core_axis_name="c", subcore_axis_name="s")
    f = functools.partial(
        pl.kernel,
        out_type=jax.ShapeDtypeStruct((_NP, _C), jnp.float32),
        mesh=mesh,
        scratch_types=[pltpu.VMEM((_PCH,), jnp.int32),
                       pltpu.VMEM((_PCH, _C), jnp.float32),
                       pltpu.VMEM((2, _OC, _C), jnp.float32),
                       pltpu.VMEM_SHARED((16, 1, 16), jnp.int32),
                       pltpu.VMEM_SHARED((16, _PCH), jnp.int32),
                       pltpu.SMEM((1, 16), jnp.int32),
                       pltpu.SMEM((_PCH,), jnp.int32),
                       pltpu.SemaphoreType.DMA,
                       pltpu.SemaphoreType.DMA,
                       pltpu.SemaphoreType.DMA],
    )(_gather_body)
    return f(tab, b0, srcs, dsts, cnts)


def kernel(features, coords, W1, W1_2, W2, W3,
           bn0_w, bn0_b, bn0_2_w, bn0_2_b, bn1_w, bn1_b, bn2_w, bn2_b):
    f32 = jnp.float32
    x = jnp.zeros((_NP, _C), f32).at[:_N].set(features)
    grid = jnp.full(_GRID, -1, jnp.int32).at[
        coords[:, 0], coords[:, 1], coords[:, 2]].set(jnp.arange(_N, dtype=jnp.int32))
    nb_a = _nbr(grid, coords, _OFF_A)
    nb_b = _nbr(grid, coords, _OFF_B)
    mult = (nb_a[4] == jnp.arange(_NP, dtype=jnp.int32)).astype(f32)[:, None]
    srcs_a, dsts_a, cnt_a = _pairs(nb_a)
    srcs_b, dsts_b, cnt_b = _pairs(nb_b)

    w1c, w12c, w2c, w3c = _cat9(W1), _cat9(W1_2), _cat9(W2), _cat9(W3)
    bn = lambda v: v.reshape(1, _C)

    y1, b01 = _mm_plain(x, w1c, mult)
    z1 = _gather_sc(y1.reshape(_NP * _K, _C), b01, srcs_a, dsts_a, cnt_a)
    p1 = _stats(z1)
    y2, b02 = _mm_fused(z1, w12c, bn(bn0_w), bn(bn0_b), p1, mult)
    z2 = _gather_sc(y2.reshape(_NP * _K, _C), b02, srcs_b, dsts_b, cnt_b)
    p2 = _stats(z2)

    y3, b03 = _mm_plain(x, w2c, mult)
    z3 = _gather_sc(y3.reshape(_NP * _K, _C), b03, srcs_b, dsts_b, cnt_b)
    p3 = _stats(z3)
    y4, b04 = _mm_fused(z3, w3c, bn(bn1_w), bn(bn1_b), p3, mult)
    z4 = _gather_sc(y4.reshape(_NP * _K, _C), b04, srcs_a, dsts_a, cnt_a)
    p4 = _stats(z4)

    out = _final(z2, p2, bn(bn0_2_w), bn(bn0_2_b), z4, p4, bn(bn2_w), bn(bn2_b))
    return out[:_N]


# speedup vs baseline: 2.4425x; 2.4425x over previous
import functools

import jax
import jax.numpy as jnp
from jax import lax
from jax.experimental import pallas as pl
from jax.experimental.pallas import tpu as pltpu
from jax.experimental.pallas import tpu_sc as plsc

_GRID = (128, 128, 128)
_N = 50000
_NW = 32
_RPW = 1600
_NP = _NW * _RPW
_C = 128
_K = 9
_OC = 320
_NOC = _RPW // _OC
_NB = _NW * _NOC
_BKT = _OC * _K
_PCH = 128
_CAPW = 3072
_W2 = 5888
_BM = 512
_NPB = _NP // _BM

_OFF_A = tuple((0, dy, dx) for dy in (-1, 0, 1) for dx in (-1, 0, 1))
_OFF_B = tuple((dz, 0, dx) for dz in (-1, 0, 1) for dx in (-1, 0, 1))


def _nbr(grid, coords, offsets):
    D, H, Wd = _GRID
    z, y, x = coords[:, 0], coords[:, 1], coords[:, 2]
    cols = []
    for oz, oy, ox in offsets:
        nz, ny, nx = z + oz, y + oy, x + ox
        valid = (nz >= 0) & (nz < D) & (ny >= 0) & (ny < H) & (nx >= 0) & (nx < Wd)
        nb = grid[jnp.clip(nz, 0, D - 1), jnp.clip(ny, 0, H - 1), jnp.clip(nx, 0, Wd - 1)]
        cols.append(jnp.where(valid & (nb >= 0), nb, _N))
    nb = jnp.stack(cols)
    pad = jnp.full((_K, _NP - _N), _N, jnp.int32)
    return jnp.concatenate([nb, pad], axis=1)


def _pairs(nb):
    nbt = nb.T
    ii = jnp.arange(_NP, dtype=jnp.int32)[:, None]
    kk = jnp.arange(_K, dtype=jnp.int32)[None, :]
    need = (nbt != _N) & ((kk != 4) | (nbt != ii))
    src = nbt * _K + kk
    dst = jnp.broadcast_to(ii % _OC, (_NP, _K))
    needb = need.reshape(_NB, _BKT)
    posb = jnp.broadcast_to(jnp.arange(_BKT, dtype=jnp.int32)[None, :],
                            (_NB, _BKT))
    key = jnp.where(needb, posb, _BKT + posb)
    srcm = jnp.where(needb, src.reshape(_NB, _BKT), _N * _K)
    dstm = jnp.where(needb, dst.reshape(_NB, _BKT), 0)
    _, srcs, dsts = lax.sort((key, srcm, dstm), dimension=1, num_keys=1)
    srcs = jnp.concatenate(
        [srcs, jnp.full((_NB, _CAPW - _BKT), _N * _K, jnp.int32)], axis=1)
    dsts = jnp.concatenate(
        [dsts, jnp.zeros((_NB, _CAPW - _BKT), jnp.int32)], axis=1)
    counts = jnp.sum(needb, axis=1, dtype=jnp.int32).reshape(_NW, _NOC)
    counts = jnp.pad(counts, ((0, 0), (0, 16 - _NOC)))
    return (srcs.reshape(_NB, 1, _CAPW), dsts.reshape(_NB, 1, _CAPW),
            counts.reshape(_NW, 1, 16))


def _cat9(W):
    return W.reshape(_K, _C, _C).transpose(1, 0, 2).reshape(_C, _K * _C)


def _mm_plain_body(x_ref, w_ref, m_ref, y_ref, b0_ref):
    y = jnp.dot(x_ref[...], w_ref[...], preferred_element_type=jnp.float32)
    y_ref[...] = y
    b0_ref[...] = y[:, 4 * _C:5 * _C] * m_ref[...]


def _mm_plain(x, wcat, mult):
    return pl.pallas_call(
        _mm_plain_body,
        grid=(_NPB,),
        in_specs=[pl.BlockSpec((_BM, _C), lambda i: (i, 0)),
                  pl.BlockSpec((_C, _K * _C), lambda i: (0, 0)),
                  pl.BlockSpec((_BM, 1), lambda i: (i, 0))],
        out_specs=[pl.BlockSpec((_BM, _K * _C), lambda i: (i, 0)),
                   pl.BlockSpec((_BM, _C), lambda i: (i, 0))],
        out_shape=[jax.ShapeDtypeStruct((_NP, _K * _C), jnp.float32),
                   jax.ShapeDtypeStruct((_NP, _C), jnp.float32)],
    )(x, wcat, mult)


def _bn_affine(p, bnw, bnb):
    psum = jnp.sum(p, axis=0)
    mu = psum[0:1] / float(_N)
    msq = psum[1:2] / float(_N)
    var = msq - mu * mu
    s = bnw * lax.rsqrt(var + 1e-5)
    t = bnb - mu * s
    return s, t


def _mm_fused_body(z_ref, w_ref, bnw_ref, bnb_ref, p_ref, m_ref, y_ref, b0_ref):
    s, t = _bn_affine(p_ref[...], bnw_ref[...], bnb_ref[...])
    z = z_ref[...]
    l = jnp.where(z >= 0.0, z, z * 0.01)
    rows = pl.program_id(0) * _BM + lax.broadcasted_iota(jnp.int32, (_BM, 1), 0)
    a = jnp.where(rows < _N, l * s + t, 0.0)
    y = jnp.dot(a, w_ref[...], preferred_element_type=jnp.float32)
    y_ref[...] = y
    b0_ref[...] = y[:, 4 * _C:5 * _C] * m_ref[...]


def _mm_fused(z, wcat, bnw, bnb, p, mult):
    return pl.pallas_call(
        _mm_fused_body,
        grid=(_NPB,),
        in_specs=[pl.BlockSpec((_BM, _C), lambda i: (i, 0)),
                  pl.BlockSpec((_C, _K * _C), lambda i: (0, 0)),
                  pl.BlockSpec((1, _C), lambda i: (0, 0)),
                  pl.BlockSpec((1, _C), lambda i: (0, 0)),
                  pl.BlockSpec((_NPB, 2, _C), lambda i: (0, 0, 0)),
                  pl.BlockSpec((_BM, 1), lambda i: (i, 0))],
        out_specs=[pl.BlockSpec((_BM, _K * _C), lambda i: (i, 0)),
                   pl.BlockSpec((_BM, _C), lambda i: (i, 0))],
        out_shape=[jax.ShapeDtypeStruct((_NP, _K * _C), jnp.float32),
                   jax.ShapeDtypeStruct((_NP, _C), jnp.float32)],
    )(z, wcat, bnw, bnb, p, mult)


def _stats_body(z_ref, p_ref):
    z = z_ref[...]
    l = jnp.where(z >= 0.0, z, z * 0.01)
    p_ref[...] = jnp.stack([jnp.sum(l, axis=0), jnp.sum(l * l, axis=0)])[None]


def _stats(z):
    return pl.pallas_call(
        _stats_body,
        grid=(_NPB,),
        in_specs=[pl.BlockSpec((_BM, _C), lambda i: (i, 0))],
        out_specs=pl.BlockSpec((1, 2, _C), lambda i: (i, 0, 0)),
        out_shape=jax.ShapeDtypeStruct((_NPB, 2, _C), jnp.float32),
    )(z)


def _final_body(z2_ref, z4_ref, p2_ref, p4_ref, w2_ref, b2_ref, w4_ref, b4_ref,
                o_ref):
    s2, t2 = _bn_affine(p2_ref[...], w2_ref[...], b2_ref[...])
    s4, t4 = _bn_affine(p4_ref[...], w4_ref[...], b4_ref[...])
    z2 = z2_ref[...]
    z4 = z4_ref[...]
    l2 = jnp.where(z2 >= 0.0, z2, z2 * 0.01)
    l4 = jnp.where(z4 >= 0.0, z4, z4 * 0.01)
    o_ref[...] = (l2 * s2 + t2) + (l4 * s4 + t4)


def _final(z2, p2, w2, b2, z4, p4, w4, b4):
    return pl.pallas_call(
        _final_body,
        grid=(_NPB,),
        in_specs=[pl.BlockSpec((_BM, _C), lambda i: (i, 0)),
                  pl.BlockSpec((_BM, _C), lambda i: (i, 0)),
                  pl.BlockSpec((_NPB, 2, _C), lambda i: (0, 0, 0)),
                  pl.BlockSpec((_NPB, 2, _C), lambda i: (0, 0, 0)),
                  pl.BlockSpec((1, _C), lambda i: (0, 0)),
                  pl.BlockSpec((1, _C), lambda i: (0, 0)),
                  pl.BlockSpec((1, _C), lambda i: (0, 0)),
                  pl.BlockSpec((1, _C), lambda i: (0, 0))],
        out_specs=pl.BlockSpec((_BM, _C), lambda i: (i, 0)),
        out_shape=jax.ShapeDtypeStruct((_NP, _C), jnp.float32),
    )(z2, z4, p2, p4, w2, b2, w4, b4)


def _gather_body(tab_hbm, b0_hbm, srcs_hbm, dsts_hbm, cnts_hbm, z_hbm,
                 srcs_v, rows_v, out_v, sh_cnt, sh_dst, cnt_sm, dst_sm,
                 semg, sw0, sw1):
    sid = lax.axis_index("s")
    wid = sid * 2 + lax.axis_index("c")
    pltpu.sync_copy(cnts_hbm.at[wid], sh_cnt.at[sid])
    pltpu.sync_copy(sh_cnt.at[sid], cnt_sm)
    sw = (sw0, sw1)
    for oc in range(_NOC):
        par = oc % 2
        obase = wid * _RPW + oc * _OC
        if oc >= 2:
            pltpu.make_async_copy(out_v.at[par], z_hbm.at[pl.ds(0, _OC)],
                                  sw[par]).wait()
        pltpu.sync_copy(b0_hbm.at[pl.ds(obase, _OC)], out_v.at[par])
        n = cnt_sm[0, oc]
        nch = lax.div(n + _PCH - 1, _PCH)

        def chunk(cc, carry, oc=oc, par=par):
            pltpu.sync_copy(srcs_hbm.at[wid * _NOC + oc, 0, pl.ds(cc * _PCH, _PCH)], srcs_v)
            pltpu.sync_copy(dsts_hbm.at[wid * _NOC + oc, 0, pl.ds(cc * _PCH, _PCH)],
                            sh_dst.at[sid])
            pltpu.sync_copy(sh_dst.at[sid], dst_sm)
            pltpu.async_copy(tab_hbm.at[srcs_v], rows_v, semg).wait()

            def pr(j, c2):
                r = dst_sm[j]
                for c in range(8):
                    sl = pl.ds(c * 16, 16)
                    plsc.addupdate(out_v.at[par, r, sl], rows_v[j, sl])
                return c2

            lax.fori_loop(0, _PCH, pr, 0)
            return carry

        lax.fori_loop(0, nch, chunk, 0)
        pltpu.async_copy(out_v.at[par], z_hbm.at[pl.ds(obase, _OC)], sw[par])
    pltpu.make_async_copy(out_v.at[1], z_hbm.at[pl.ds(0, _OC)], sw1).wait()
    pltpu.make_async_copy(out_v.at[0], z_hbm.at[pl.ds(0, _OC)], sw0).wait()


def _gather_sc(tab, b0, srcs, dsts, cnts):
    mesh = plsc.VectorSubcoreMesh(core_axis_name="c", subcore_axis_name="s")
    f = functools.partial(
        pl.kernel,
        out_type=jax.ShapeDtypeStruct((_NP, _C), jnp.float32),
        mesh=mesh,
        scratch_types=[pltpu.VMEM((_PCH,), jnp.int32),
                       pltpu.VMEM((_PCH, _C), jnp.float32),
                       pltpu.VMEM((2, _OC, _C), jnp.float32),
                       pltpu.VMEM_SHARED((16, 1, 16), jnp.int32),
                       pltpu.VMEM_SHARED((16, _PCH), jnp.int32),
                       pltpu.SMEM((1, 16), jnp.int32),
                       pltpu.SMEM((_PCH,), jnp.int32),
                       pltpu.SemaphoreType.DMA,
                       pltpu.SemaphoreType.DMA,
                       pltpu.SemaphoreType.DMA],
    )(_gather_body)
    return f(tab, b0, srcs, dsts, cnts)


def kernel(features, coords, W1, W1_2, W2, W3,
           bn0_w, bn0_b, bn0_2_w, bn0_2_b, bn1_w, bn1_b, bn2_w, bn2_b):
    f32 = jnp.float32
    x = jnp.zeros((_NP, _C), f32).at[:_N].set(features)
    grid = jnp.full(_GRID, -1, jnp.int32).at[
        coords[:, 0], coords[:, 1], coords[:, 2]].set(jnp.arange(_N, dtype=jnp.int32))
    nb_a = _nbr(grid, coords, _OFF_A)
    nb_b = _nbr(grid, coords, _OFF_B)
    mult = (nb_a[4] == jnp.arange(_NP, dtype=jnp.int32)).astype(f32)[:, None]
    srcs_a, dsts_a, cnt_a = _pairs(nb_a)
    srcs_b, dsts_b, cnt_b = _pairs(nb_b)

    w1c, w12c, w2c, w3c = _cat9(W1), _cat9(W1_2), _cat9(W2), _cat9(W3)
    bn = lambda v: v.reshape(1, _C)

    y1, b01 = _mm_plain(x, w1c, mult)
    z1 = _gather_sc(y1.reshape(_NP * _K, _C), b01, srcs_a, dsts_a, cnt_a)
    p1 = _stats(z1)
    y2, b02 = _mm_fused(z1, w12c, bn(bn0_w), bn(bn0_b), p1, mult)
    z2 = _gather_sc(y2.reshape(_NP * _K, _C), b02, srcs_b, dsts_b, cnt_b)
    p2 = _stats(z2)

    y3, b03 = _mm_plain(x, w2c, mult)
    z3 = _gather_sc(y3.reshape(_NP * _K, _C), b03, srcs_b, dsts_b, cnt_b)
    p3 = _stats(z3)
    y4, b04 = _mm_fused(z3, w3c, bn(bn1_w), bn(bn1_b), p3, mult)
    z4 = _gather_sc(y4.reshape(_NP * _K, _C), b04, srcs_a, dsts_a, cnt_a)
    p4 = _stats(z4)

    out = _final(z2, p2, bn(bn0_2_w), bn(bn0_2_b), z4, p4, bn(bn2_w), bn(bn2_b))
    return out[:_N]
